# baseline (speedup 1.0000x reference)
import functools

import jax
import jax.numpy as jnp
from jax import lax
from jax.experimental import pallas as pl
from jax.experimental.pallas import tpu as pltpu
from jax.experimental.pallas import tpu_sc as plsc

N = 10000
D = 256
C = 7
A4 = 0.1

NC = 2
NS = 16
NW = NC * NS
CP = 8
CH = 80
NCHUNK = N // CH
MAXK = -(-NCHUNK // NW)
_FULL = NCHUNK - (NCHUNK // NW) * NW

LBLK = 16


def _ew_body(seq_ref, seq1_ref, p1_ref, p2_ref, p3_ref, wp_ref, dff_ref,
             dp_ref, raw_ref):
    wp0 = wp_ref[0, 0]
    wp1 = wp_ref[0, 1]
    wp2 = wp_ref[0, 2]
    d0 = dff_ref[0, 0]
    d1 = dff_ref[0, 1]
    t = wp0 * p1_ref[...] + wp1 * p2_ref[...] + wp2 * p3_ref[...]
    te = jnp.exp(jnp.minimum(t, 0.0)) - 1.0
    wvec = 1.0 + jnp.where(t > 0.0, t, te)
    s = d0 * wvec + d1 * dp_ref[...]

    x = seq_ref[...]
    y = x * s
    e = jnp.exp(jnp.minimum(y, 0.0)) - 1.0
    raw_ref[...] = jnp.where(y > 0.0, y, e) + A4 * seq1_ref[...]


def _stage_a(seq, seq1, p1, p2, p3, wp_weight, dff_weight, dp_weight):
    blk = 1000
    row = pl.BlockSpec((1, D), lambda i: (0, 0))
    return pl.pallas_call(
        _ew_body,
        grid=(N // blk,),
        in_specs=[
            pl.BlockSpec((blk, D), lambda i: (i, 0)),
            pl.BlockSpec((blk, D), lambda i: (i, 0)),
            row, row, row,
            pl.BlockSpec((1, 3), lambda i: (0, 0)),
            pl.BlockSpec((1, 2), lambda i: (0, 0)),
            row,
        ],
        out_specs=pl.BlockSpec((blk, D), lambda i: (i, 0)),
        out_shape=jax.ShapeDtypeStruct((N, D), jnp.float32),
    )(seq, seq1, p1, p2, p3, wp_weight, dff_weight, dp_weight)


def _sc_body(raw_hbm, lab_hbm,
             sums_out,
             raw0, raw1, lab0, lab1, sums_v,
             sr0, sr1, sl0, sl1):
    wid = lax.axis_index("s") * NC + lax.axis_index("c")
    raws = (raw0, raw1)
    labs = (lab0, lab1)
    srs = (sr0, sr1)
    sls = (sl0, sl1)
    lanes = lax.iota(jnp.int32, 16)
    zero16 = jnp.zeros((16,), jnp.float32)

    for c in range(CP):
        for t in range(D // 16):
            sums_v[c, pl.ds(t * 16, 16)] = zero16

    def issue(i):
        slot = i % 2
        base = (wid + NW * i) * CH
        pltpu.async_copy(raw_hbm.at[pl.ds(base, CH)], raws[slot], srs[slot])
        pltpu.async_copy(lab_hbm.at[pl.ds(base, CH)], labs[slot], sls[slot])

    def wait(i):
        slot = i % 2
        base = (wid + NW * i) * CH
        pltpu.make_async_copy(raw_hbm.at[pl.ds(base, CH)],
                              raws[slot], srs[slot]).wait()
        pltpu.make_async_copy(lab_hbm.at[pl.ds(base, CH)], labs[slot],
                              sls[slot]).wait()

    def process(i):
        slot = i % 2
        raw_v = raws[slot]
        lab_v = labs[slot]

        @plsc.parallel_loop(0, CH, 1, unroll=2)
        def _(r):
            lvec = plsc.load_gather(lab_v.at[...], [jnp.full((16,), r)])
            for j in range(D // 16):
                z = raw_v[r, pl.ds(j * 16, 16)]
                plsc.addupdate_scatter(sums_v.at[...],
                                       [lvec, lanes + j * 16], z)

    issue(0)
    for i in range(MAXK - 1):
        wait(i)
        if i + 1 < MAXK - 1:
            issue(i + 1)
        else:
            @pl.when(wid < _FULL)
            def _():
                issue(MAXK - 1)
        process(i)

    @pl.when(wid < _FULL)
    def _():
        wait(MAXK - 1)
        process(MAXK - 1)

    pltpu.sync_copy(sums_v, sums_out.at[pl.ds(wid * CP, CP)])


_stage_b = functools.partial(
    pl.kernel,
    out_type=[
        jax.ShapeDtypeStruct((NW * CP, D), jnp.float32),
    ],
    mesh=plsc.VectorSubcoreMesh(core_axis_name="c", subcore_axis_name="s"),
    compiler_params=pltpu.CompilerParams(needs_layout_passes=False),
    scratch_types=[
        pltpu.VMEM((CH, D), jnp.float32),
        pltpu.VMEM((CH, D), jnp.float32),
        pltpu.VMEM((CH,), jnp.int32),
        pltpu.VMEM((CH,), jnp.int32),
        pltpu.VMEM((CP, D), jnp.float32),
        pltpu.SemaphoreType.DMA,
        pltpu.SemaphoreType.DMA,
        pltpu.SemaphoreType.DMA,
        pltpu.SemaphoreType.DMA,
    ],
)(_sc_body)


def _head_body(raw_ref, sums_ref, lab_ref, out_ref, ave_ref):
    @pl.when(pl.program_id(0) == 0)
    def _():
        s32 = sums_ref[...]
        labs = lab_ref[...]
        sums = s32[0:C, :]
        for w in range(1, NW):
            sums = sums + s32[w * CP:w * CP + C, :]
        ci = lax.broadcasted_iota(jnp.int32, (C, 1), 0)
        counts = jnp.zeros((C, 1), jnp.float32)
        for c in range(C):
            cc = jnp.sum(jnp.where(labs == c, 1.0, 0.0))
            counts = counts + jnp.where(ci == c, cc, 0.0)
        ave = sums / jnp.maximum(counts, 1.0)
        an = jnp.maximum(jnp.sqrt(jnp.sum(ave * ave, axis=1, keepdims=True)),
                         1e-8)
        ave_ref[...] = ave / an

    raw = raw_ref[...]
    ave = ave_ref[...]
    ones8 = jnp.ones((8, D), jnp.float32)
    rn2_t = lax.dot_general(ones8, raw * raw, (((1,), (1,)), ((), ())),
                            preferred_element_type=jnp.float32)
    rn_t = jnp.maximum(jnp.sqrt(rn2_t[0:1, :]), 1e-8)
    ret_t = lax.dot_general(ave, raw, (((1,), (1,)), ((), ())),
                            preferred_element_type=jnp.float32)
    ret_t = ret_t / rn_t
    m = jnp.max(ret_t, axis=0, keepdims=True)
    e = jnp.exp(ret_t - m)
    sm_t = e / jnp.sum(e, axis=0, keepdims=True)
    out_ref[...] = sm_t.T


def _stage_c(raw, sums32, lab2d):
    blk = 1000
    return pl.pallas_call(
        _head_body,
        grid=(N // blk,),
        in_specs=[
            pl.BlockSpec((blk, D), lambda i: (i, 0)),
            pl.BlockSpec((NW * CP, D), lambda i: (0, 0)),
            pl.BlockSpec((N // LBLK, LBLK), lambda i: (0, 0)),
        ],
        out_specs=pl.BlockSpec((blk, C), lambda i: (i, 0)),
        out_shape=jax.ShapeDtypeStruct((N, C), jnp.float32),
        scratch_shapes=[
            pltpu.VMEM((C, D), jnp.float32),
        ],
    )(raw, sums32, lab2d)


def kernel(seq, seq1, labels, prompt1, prompt2, prompt3,
           wp_weight, dff_weight, dp_weight):
    raw = _stage_a(seq, seq1, prompt1, prompt2, prompt3,
                   wp_weight, dff_weight, dp_weight)
    (sums32,) = _stage_b(raw, labels)
    return _stage_c(raw, sums32, labels.reshape(N // LBLK, LBLK))


# speedup vs baseline: 1.0637x; 1.0637x over previous
import functools

import jax
import jax.numpy as jnp
from jax import lax
from jax.experimental import pallas as pl
from jax.experimental.pallas import tpu as pltpu
from jax.experimental.pallas import tpu_sc as plsc

N = 10000
D = 256
C = 7
A4 = 0.1

NC = 2
NS = 16
NW = NC * NS
CP = 8
CH = 80
NCHUNK = N // CH
MAXK = -(-NCHUNK // NW)
_FULL = NCHUNK - (NCHUNK // NW) * NW

LBLK = 16


def _ew_body(seq_ref, seq1_ref, p1_ref, p2_ref, p3_ref, wp_ref, dff_ref,
             dp_ref, raw_ref):
    wp0 = wp_ref[0, 0]
    wp1 = wp_ref[0, 1]
    wp2 = wp_ref[0, 2]
    d0 = dff_ref[0, 0]
    d1 = dff_ref[0, 1]
    t = wp0 * p1_ref[...] + wp1 * p2_ref[...] + wp2 * p3_ref[...]
    te = jnp.exp(jnp.minimum(t, 0.0)) - 1.0
    wvec = 1.0 + jnp.where(t > 0.0, t, te)
    s = d0 * wvec + d1 * dp_ref[...]

    x = seq_ref[...]
    y = x * s
    e = jnp.exp(jnp.minimum(y, 0.0)) - 1.0
    raw_ref[...] = jnp.where(y > 0.0, y, e) + A4 * seq1_ref[...]


def _stage_a(seq, seq1, p1, p2, p3, wp_weight, dff_weight, dp_weight):
    blk = 2000
    row = pl.BlockSpec((1, D), lambda i: (0, 0))
    return pl.pallas_call(
        _ew_body,
        grid=(N // blk,),
        in_specs=[
            pl.BlockSpec((blk, D), lambda i: (i, 0)),
            pl.BlockSpec((blk, D), lambda i: (i, 0)),
            row, row, row,
            pl.BlockSpec((1, 3), lambda i: (0, 0)),
            pl.BlockSpec((1, 2), lambda i: (0, 0)),
            row,
        ],
        out_specs=pl.BlockSpec((blk, D), lambda i: (i, 0)),
        out_shape=jax.ShapeDtypeStruct((N, D), jnp.float32),
    )(seq, seq1, p1, p2, p3, wp_weight, dff_weight, dp_weight)


def _sc_body(raw_hbm, lab_hbm,
             sums_out,
             raw0, raw1, lab0, lab1, sums_v,
             sr0, sr1, sl0, sl1):
    wid = lax.axis_index("s") * NC + lax.axis_index("c")
    raws = (raw0, raw1)
    labs = (lab0, lab1)
    srs = (sr0, sr1)
    sls = (sl0, sl1)
    lanes = lax.iota(jnp.int32, 16)
    zero16 = jnp.zeros((16,), jnp.float32)

    for c in range(CP):
        for t in range(D // 16):
            sums_v[c, pl.ds(t * 16, 16)] = zero16

    def issue(i):
        slot = i % 2
        base = (wid + NW * i) * CH
        pltpu.async_copy(raw_hbm.at[pl.ds(base, CH)], raws[slot], srs[slot])
        pltpu.async_copy(lab_hbm.at[pl.ds(base, CH)], labs[slot], sls[slot])

    def wait(i):
        slot = i % 2
        base = (wid + NW * i) * CH
        pltpu.make_async_copy(raw_hbm.at[pl.ds(base, CH)],
                              raws[slot], srs[slot]).wait()
        pltpu.make_async_copy(lab_hbm.at[pl.ds(base, CH)], labs[slot],
                              sls[slot]).wait()

    def process(i):
        slot = i % 2
        raw_v = raws[slot]
        lab_v = labs[slot]

        @plsc.parallel_loop(0, CH, 1, unroll=2)
        def _(r):
            lvec = plsc.load_gather(lab_v.at[...], [jnp.full((16,), r)])
            for j in range(D // 16):
                z = raw_v[r, pl.ds(j * 16, 16)]
                plsc.addupdate_scatter(sums_v.at[...],
                                       [lvec, lanes + j * 16], z)

    issue(0)
    for i in range(MAXK - 1):
        wait(i)
        if i + 1 < MAXK - 1:
            issue(i + 1)
        else:
            @pl.when(wid < _FULL)
            def _():
                issue(MAXK - 1)
        process(i)

    @pl.when(wid < _FULL)
    def _():
        wait(MAXK - 1)
        process(MAXK - 1)

    pltpu.sync_copy(sums_v, sums_out.at[pl.ds(wid * CP, CP)])


_stage_b = functools.partial(
    pl.kernel,
    out_type=[
        jax.ShapeDtypeStruct((NW * CP, D), jnp.float32),
    ],
    mesh=plsc.VectorSubcoreMesh(core_axis_name="c", subcore_axis_name="s"),
    compiler_params=pltpu.CompilerParams(needs_layout_passes=False),
    scratch_types=[
        pltpu.VMEM((CH, D), jnp.float32),
        pltpu.VMEM((CH, D), jnp.float32),
        pltpu.VMEM((CH,), jnp.int32),
        pltpu.VMEM((CH,), jnp.int32),
        pltpu.VMEM((CP, D), jnp.float32),
        pltpu.SemaphoreType.DMA,
        pltpu.SemaphoreType.DMA,
        pltpu.SemaphoreType.DMA,
        pltpu.SemaphoreType.DMA,
    ],
)(_sc_body)


def _head_body(raw_ref, sums_ref, lab_ref, out_ref, ave_ref):
    @pl.when(pl.program_id(0) == 0)
    def _():
        s32 = sums_ref[...]
        labs = lab_ref[...]
        sums = s32[0:C, :]
        for w in range(1, NW):
            sums = sums + s32[w * CP:w * CP + C, :]
        ci = lax.broadcasted_iota(jnp.int32, (C, 1), 0)
        counts = jnp.zeros((C, 1), jnp.float32)
        for c in range(C):
            cc = jnp.sum(jnp.where(labs == c, 1.0, 0.0))
            counts = counts + jnp.where(ci == c, cc, 0.0)
        ave = sums / jnp.maximum(counts, 1.0)
        an = jnp.maximum(jnp.sqrt(jnp.sum(ave * ave, axis=1, keepdims=True)),
                         1e-8)
        ave_ref[...] = ave / an

    raw = raw_ref[...]
    ave = ave_ref[...]
    ones8 = jnp.ones((8, D), jnp.float32)
    rn2_t = lax.dot_general(ones8, raw * raw, (((1,), (1,)), ((), ())),
                            preferred_element_type=jnp.float32)
    rn_t = jnp.maximum(jnp.sqrt(rn2_t[0:1, :]), 1e-8)
    ret_t = lax.dot_general(ave, raw, (((1,), (1,)), ((), ())),
                            preferred_element_type=jnp.float32)
    ret_t = ret_t / rn_t
    m = jnp.max(ret_t, axis=0, keepdims=True)
    e = jnp.exp(ret_t - m)
    sm_t = e / jnp.sum(e, axis=0, keepdims=True)
    out_ref[...] = sm_t.T


def _stage_c(raw, sums32, lab2d):
    blk = 2000
    return pl.pallas_call(
        _head_body,
        grid=(N // blk,),
        in_specs=[
            pl.BlockSpec((blk, D), lambda i: (i, 0)),
            pl.BlockSpec((NW * CP, D), lambda i: (0, 0)),
            pl.BlockSpec((N // LBLK, LBLK), lambda i: (0, 0)),
        ],
        out_specs=pl.BlockSpec((blk, C), lambda i: (i, 0)),
        out_shape=jax.ShapeDtypeStruct((N, C), jnp.float32),
        scratch_shapes=[
            pltpu.VMEM((C, D), jnp.float32),
        ],
    )(raw, sums32, lab2d)


def kernel(seq, seq1, labels, prompt1, prompt2, prompt3,
           wp_weight, dff_weight, dp_weight):
    raw = _stage_a(seq, seq1, prompt1, prompt2, prompt3,
                   wp_weight, dff_weight, dp_weight)
    (sums32,) = _stage_b(raw, labels)
    return _stage_c(raw, sums32, labels.reshape(N // LBLK, LBLK))


# speedup vs baseline: 1.1406x; 1.0724x over previous
import functools

import jax
import jax.numpy as jnp
from jax import lax
from jax.experimental import pallas as pl
from jax.experimental.pallas import tpu as pltpu
from jax.experimental.pallas import tpu_sc as plsc

N = 10000
D = 256
C = 7
A4 = 0.1

NC = 2
NS = 16
NW = NC * NS
CP = 8
CH = 80
NCHUNK = N // CH
MAXK = -(-NCHUNK // NW)
_FULL = NCHUNK - (NCHUNK // NW) * NW

LBLK = 16


def _ew_body(seq_ref, seq1_ref, p1_ref, p2_ref, p3_ref, wp_ref, dff_ref,
             dp_ref, raw_ref):
    wp0 = wp_ref[0, 0]
    wp1 = wp_ref[0, 1]
    wp2 = wp_ref[0, 2]
    d0 = dff_ref[0, 0]
    d1 = dff_ref[0, 1]
    t = wp0 * p1_ref[...] + wp1 * p2_ref[...] + wp2 * p3_ref[...]
    te = jnp.exp(jnp.minimum(t, 0.0)) - 1.0
    wvec = 1.0 + jnp.where(t > 0.0, t, te)
    s = d0 * wvec + d1 * dp_ref[...]

    x = seq_ref[...]
    y = x * s
    e = jnp.exp(jnp.minimum(y, 0.0)) - 1.0
    raw_ref[...] = jnp.where(y > 0.0, y, e) + A4 * seq1_ref[...]


def _stage_a(seq, seq1, p1, p2, p3, wp_weight, dff_weight, dp_weight):
    blk = 2000
    row = pl.BlockSpec((1, D), lambda i: (0, 0))
    return pl.pallas_call(
        _ew_body,
        grid=(N // blk,),
        in_specs=[
            pl.BlockSpec((blk, D), lambda i: (i, 0)),
            pl.BlockSpec((blk, D), lambda i: (i, 0)),
            row, row, row,
            pl.BlockSpec((1, 3), lambda i: (0, 0)),
            pl.BlockSpec((1, 2), lambda i: (0, 0)),
            row,
        ],
        out_specs=pl.BlockSpec((blk, D), lambda i: (i, 0)),
        out_shape=jax.ShapeDtypeStruct((N, D), jnp.float32),
    )(seq, seq1, p1, p2, p3, wp_weight, dff_weight, dp_weight)


def _sc_body(raw_hbm, lab_hbm,
             sums_out,
             raw0, raw1, lab0, lab1, sums_v,
             sr0, sr1, sl0, sl1):
    wid = lax.axis_index("s") * NC + lax.axis_index("c")
    raws = (raw0, raw1)
    labs = (lab0, lab1)
    srs = (sr0, sr1)
    sls = (sl0, sl1)
    lanes = lax.iota(jnp.int32, 16)
    zero16 = jnp.zeros((16,), jnp.float32)

    for c in range(CP):
        for t in range(D // 16):
            sums_v[c, pl.ds(t * 16, 16)] = zero16

    def issue(i):
        slot = i % 2
        base = (wid + NW * i) * CH
        pltpu.async_copy(raw_hbm.at[pl.ds(base, CH)], raws[slot], srs[slot])
        pltpu.async_copy(lab_hbm.at[pl.ds(base, CH)], labs[slot], sls[slot])

    def wait(i):
        slot = i % 2
        base = (wid + NW * i) * CH
        pltpu.make_async_copy(raw_hbm.at[pl.ds(base, CH)],
                              raws[slot], srs[slot]).wait()
        pltpu.make_async_copy(lab_hbm.at[pl.ds(base, CH)], labs[slot],
                              sls[slot]).wait()

    def process(i):
        slot = i % 2
        raw_v = raws[slot]
        lab_v = labs[slot]

        @plsc.parallel_loop(0, CH, 1, unroll=2)
        def _(r):
            lvec = plsc.load_gather(lab_v.at[...], [jnp.full((16,), r)])
            for j in range(D // 16):
                z = raw_v[r, pl.ds(j * 16, 16)]
                plsc.addupdate_scatter(sums_v.at[...],
                                       [lvec, lanes + j * 16], z)

    issue(0)
    for i in range(MAXK - 1):
        wait(i)
        if i + 1 < MAXK - 1:
            issue(i + 1)
        else:
            @pl.when(wid < _FULL)
            def _():
                issue(MAXK - 1)
        process(i)

    @pl.when(wid < _FULL)
    def _():
        wait(MAXK - 1)
        process(MAXK - 1)

    pltpu.sync_copy(sums_v, sums_out.at[pl.ds(wid * CP, CP)])


_stage_b = functools.partial(
    pl.kernel,
    out_type=[
        jax.ShapeDtypeStruct((NW * CP, D), jnp.float32),
    ],
    mesh=plsc.VectorSubcoreMesh(core_axis_name="c", subcore_axis_name="s"),
    compiler_params=pltpu.CompilerParams(needs_layout_passes=False),
    scratch_types=[
        pltpu.VMEM((CH, D), jnp.float32),
        pltpu.VMEM((CH, D), jnp.float32),
        pltpu.VMEM((CH,), jnp.int32),
        pltpu.VMEM((CH,), jnp.int32),
        pltpu.VMEM((CP, D), jnp.float32),
        pltpu.SemaphoreType.DMA,
        pltpu.SemaphoreType.DMA,
        pltpu.SemaphoreType.DMA,
        pltpu.SemaphoreType.DMA,
    ],
)(_sc_body)


def _head_body(raw_ref, sums_ref, lab_ref, out_ref, ave_ref):
    @pl.when(pl.program_id(0) == 0)
    def _():
        s32 = sums_ref[...]
        labs = lab_ref[...]
        sums = s32[0:C, :]
        for w in range(1, NW):
            sums = sums + s32[w * CP:w * CP + C, :]
        ci = lax.broadcasted_iota(jnp.int32, (C, 1), 0)
        counts = jnp.zeros((C, 1), jnp.float32)
        for c in range(C):
            cc = jnp.sum(jnp.where(labs == c, 1.0, 0.0))
            counts = counts + jnp.where(ci == c, cc, 0.0)
        ave = sums / jnp.maximum(counts, 1.0)
        an = jnp.maximum(jnp.sqrt(jnp.sum(ave * ave, axis=1, keepdims=True)),
                         1e-8)
        ave_ref[...] = ave / an

    raw = raw_ref[...]
    ave = ave_ref[...]
    ones8 = jnp.ones((8, D), jnp.float32)
    rn2_t = lax.dot_general(ones8, raw * raw, (((1,), (1,)), ((), ())),
                            preferred_element_type=jnp.float32)
    rn_t = jnp.maximum(jnp.sqrt(rn2_t[0:1, :]), 1e-8)
    ret_t = lax.dot_general(ave, raw, (((1,), (1,)), ((), ())),
                            preferred_element_type=jnp.float32)
    ret_t = ret_t / rn_t
    m = jnp.max(ret_t, axis=0, keepdims=True)
    e = jnp.exp(ret_t - m)
    sm_t = e / jnp.sum(e, axis=0, keepdims=True)
    out_ref[...] = sm_t[None]


def _stage_c(raw, sums32, lab2d):
    blk = 2000
    return pl.pallas_call(
        _head_body,
        grid=(N // blk,),
        in_specs=[
            pl.BlockSpec((blk, D), lambda i: (i, 0)),
            pl.BlockSpec((NW * CP, D), lambda i: (0, 0)),
            pl.BlockSpec((N // LBLK, LBLK), lambda i: (0, 0)),
        ],
        out_specs=pl.BlockSpec((1, C, blk), lambda i: (i, 0, 0)),
        out_shape=jax.ShapeDtypeStruct((N // blk, C, blk), jnp.float32),
        scratch_shapes=[
            pltpu.VMEM((C, D), jnp.float32),
        ],
    )(raw, sums32, lab2d)


def kernel(seq, seq1, labels, prompt1, prompt2, prompt3,
           wp_weight, dff_weight, dp_weight):
    raw = _stage_a(seq, seq1, prompt1, prompt2, prompt3,
                   wp_weight, dff_weight, dp_weight)
    (sums32,) = _stage_b(raw, labels)
    out3 = _stage_c(raw, sums32, labels.reshape(N // LBLK, LBLK))
    return jnp.swapaxes(out3, 0, 1).reshape(C, N).T
